# baseline (speedup 1.0000x reference)
import functools

import jax
import jax.numpy as jnp
from jax import lax
from jax.experimental import pallas as pl
from jax.experimental.pallas import tpu as pltpu
from jax.experimental.pallas import tpu_sc as plsc

_B = 16384
_D = 128
_CHUNK = 64
_DEPTH = 2
_NUM_CORES = 2
_NUM_SUBCORES = 16


def _build():
    nw = _NUM_CORES * _NUM_SUBCORES
    b_per_w = _B // nw
    n_chunks = b_per_w // _CHUNK
    mesh = plsc.VectorSubcoreMesh(core_axis_name="c", subcore_axis_name="s")

    @functools.partial(
        pl.kernel,
        mesh=mesh,
        out_type=jax.ShapeDtypeStruct((_B, _D), jnp.float32),
        scratch_types=[
            pltpu.VMEM((b_per_w,), jnp.int32),
            pltpu.VMEM((b_per_w, _D), jnp.float32),
        ]
        + [pltpu.SemaphoreType.DMA] * n_chunks
        + [pltpu.SemaphoreType.DMA],
    )
    def emb(labels_hbm, table_hbm, out_hbm, idx_v, rows_v, *sems):
        gsems, wsem = sems[:n_chunks], sems[n_chunks]
        wid = lax.axis_index("s") * _NUM_CORES + lax.axis_index("c")
        base = wid * b_per_w
        pltpu.sync_copy(labels_hbm.at[pl.ds(base, b_per_w)], idx_v)

        def gather(j):
            return pltpu.async_copy(
                table_hbm.at[idx_v.at[pl.ds(j * _CHUNK, _CHUNK)]],
                rows_v.at[pl.ds(j * _CHUNK, _CHUNK)],
                gsems[j],
            )

        gathers = {j: gather(j) for j in range(min(_DEPTH, n_chunks))}
        writes = []
        for j in range(n_chunks):
            gathers[j].wait()
            writes.append(
                pltpu.async_copy(
                    rows_v.at[pl.ds(j * _CHUNK, _CHUNK)],
                    out_hbm.at[pl.ds(base + j * _CHUNK, _CHUNK)],
                    wsem,
                )
            )
            nxt = j + _DEPTH
            if nxt < n_chunks:
                gathers[nxt] = gather(nxt)
        for w in writes:
            w.wait()

    return emb


_emb_cache = []


def kernel(labels, train, table):
    if not _emb_cache:
        _emb_cache.append(_build())
    return _emb_cache[0](labels.astype(jnp.int32), table)


# speedup vs baseline: 1.0659x; 1.0659x over previous
import functools

import jax
import jax.numpy as jnp
from jax import lax
from jax.experimental import pallas as pl
from jax.experimental.pallas import tpu as pltpu
from jax.experimental.pallas import tpu_sc as plsc

_B = 16384
_D = 128
_CHUNK = 128
_NUM_CORES = 2
_NUM_SUBCORES = 16


def _build():
    nw = _NUM_CORES * _NUM_SUBCORES
    b_per_w = _B // nw
    n_chunks = b_per_w // _CHUNK
    mesh = plsc.VectorSubcoreMesh(core_axis_name="c", subcore_axis_name="s")

    @functools.partial(
        pl.kernel,
        mesh=mesh,
        out_type=jax.ShapeDtypeStruct((_B, _D), jnp.float32),
        scratch_types=[
            pltpu.VMEM((b_per_w,), jnp.int32),
            pltpu.VMEM((b_per_w, _D), jnp.float32),
            pltpu.SemaphoreType.DMA,
        ],
    )
    def emb(labels_hbm, table_hbm, out_hbm, idx_v, rows_v, sem):
        wid = lax.axis_index("s") * _NUM_CORES + lax.axis_index("c")
        base = wid * b_per_w
        pltpu.sync_copy(labels_hbm.at[pl.ds(base, b_per_w)], idx_v)
        gathers = [
            pltpu.async_copy(
                table_hbm.at[idx_v.at[pl.ds(j * _CHUNK, _CHUNK)]],
                rows_v.at[pl.ds(j * _CHUNK, _CHUNK)],
                sem,
            )
            for j in range(n_chunks)
        ]
        for g in gathers:
            g.wait()
        pltpu.sync_copy(rows_v, out_hbm.at[pl.ds(base, b_per_w)])

    return emb


_emb_cache = []


def kernel(labels, train, table):
    if not _emb_cache:
        _emb_cache.append(_build())
    return _emb_cache[0](labels.astype(jnp.int32), table)
